# baseline (speedup 1.0000x reference)
import functools

import jax
import jax.numpy as jnp
from jax import lax
from jax.experimental import pallas as pl
from jax.experimental.pallas import tpu as pltpu
from jax.experimental.pallas import tpu_sc as plsc

_LANES = 16


def _sc_broadcast(bs: int, row: int):
    NC, NS = 2, 16
    NW = NC * NS
    rows_per_w = bs // NW
    C = 64
    n_dma = rows_per_w // C

    mesh = plsc.VectorSubcoreMesh(core_axis_name="c", subcore_axis_name="s")

    @functools.partial(
        pl.kernel,
        out_type=jax.ShapeDtypeStruct((bs, row), jnp.float32),
        mesh=mesh,
        scratch_types=[
            pltpu.VMEM((row,), jnp.float32),
            pltpu.VMEM((C, row), jnp.float32),
            pltpu.SemaphoreType.DMA,
        ],
    )
    def body(w_hbm, out_hbm, w_v, buf_v, sem):
        wid = lax.axis_index("s") * NC + lax.axis_index("c")
        pltpu.sync_copy(w_hbm, w_v)

        def fill_row(r, carry):
            for j in range(row // _LANES):
                buf_v[r, pl.ds(j * _LANES, _LANES)] = w_v[pl.ds(j * _LANES, _LANES)]
            return carry

        lax.fori_loop(0, C, fill_row, 0)

        base = wid * rows_per_w
        copies = [
            pltpu.async_copy(buf_v, out_hbm.at[pl.ds(base + d * C, C)], sem)
            for d in range(n_dma)
        ]
        for c in copies:
            c.wait()

    return body


def kernel(x, W):
    bs, frame_num, _ = x.shape
    num_feats = W.shape[1]
    row = frame_num * num_feats
    w_flat = W.reshape(row)
    out = _sc_broadcast(bs, row)(w_flat)
    return out.reshape(bs, frame_num, num_feats)


# speedup vs baseline: 1.0562x; 1.0562x over previous
import functools

import jax
import jax.numpy as jnp
from jax import lax
from jax.experimental import pallas as pl
from jax.experimental.pallas import tpu as pltpu
from jax.experimental.pallas import tpu_sc as plsc

_LANES = 16


def _sc_broadcast(bs: int, row: int):
    NC, NS = 2, 16
    NW = NC * NS
    rows_per_w = bs // NW
    C = 16
    n_dma = rows_per_w // C

    mesh = plsc.VectorSubcoreMesh(core_axis_name="c", subcore_axis_name="s")

    @functools.partial(
        pl.kernel,
        out_type=jax.ShapeDtypeStruct((bs, row), jnp.float32),
        mesh=mesh,
        scratch_types=[
            pltpu.VMEM((row,), jnp.float32),
            pltpu.VMEM((C, row), jnp.float32),
            pltpu.SemaphoreType.DMA,
        ],
    )
    def body(w_hbm, out_hbm, w_v, buf_v, sem):
        wid = lax.axis_index("s") * NC + lax.axis_index("c")
        pltpu.sync_copy(w_hbm, w_v)

        def fill_row(r, carry):
            for j in range(row // _LANES):
                buf_v[r, pl.ds(j * _LANES, _LANES)] = w_v[pl.ds(j * _LANES, _LANES)]
            return carry

        lax.fori_loop(0, C, fill_row, 0)

        base = wid * rows_per_w
        copies = [
            pltpu.async_copy(buf_v, out_hbm.at[pl.ds(base + d * C, C)], sem)
            for d in range(n_dma)
        ]
        for c in copies:
            c.wait()

    return body


def kernel(x, W):
    bs, frame_num, _ = x.shape
    num_feats = W.shape[1]
    row = frame_num * num_feats
    w_flat = W.reshape(row)
    out = _sc_broadcast(bs, row)(w_flat)
    return out.reshape(bs, frame_num, num_feats)
